# baseline (speedup 1.0000x reference)
import jax
import jax.numpy as jnp
from jax import lax
from jax.experimental import pallas as pl
from jax.experimental.pallas import tpu as pltpu
from jax.experimental.pallas import tpu_sc as plsc

N = 10000
D_IN = 128
D_H = 256
D_L = 64

NC = 2
NS = 16
NW = NC * NS
K = 128
NB = 176
NB2 = NB // 2
E_PAD = NW * NB2 * K
NPH = 5
NR = 2000
NBW = 88
ACC_ROWS = 10008
ZMAIN = 632
ZLAST = 528

R = 1000
G = N // R
BNC = 0.9999950000374997


def _mesh():
    return plsc.VectorSubcoreMesh(core_axis_name="c", subcore_axis_name="s")


def _prep_kernel(src2, dst2, izeros, itrash):
    def body(src_hbm, dst_hbm, iz_hbm, it_hbm, bsrc_hbm, bdst_hbm, cnt_hbm,
             in_s, in_d, st_s, st_d, cntv):
        c = lax.axis_index("c")
        s = lax.axis_index("s")
        w = c * NS + s
        for p in range(NPH):
            pltpu.sync_copy(iz_hbm, st_s.at[p, pl.ds(0, NBW)])
            pltpu.sync_copy(it_hbm, st_d.at[p, pl.ds(0, NBW)])
        iota = lax.broadcasted_iota(jnp.int32, (16,), 0)

        def chunk(t, wp):
            pltpu.sync_copy(src_hbm.at[w, pl.ds(t * 8, 8)], in_s)
            pltpu.sync_copy(dst_hbm.at[w, pl.ds(t * 8, 8)], in_d)

            def group(g, wp2):
                r = lax.shift_right_logical(g, 3)
                col = jnp.bitwise_and(g, 7) * 16
                sv = in_s[r, pl.ds(col, 16)]
                dv = in_d[r, pl.ds(col, 16)]
                new_wp = []
                for p in range(NPH):
                    lo = p * NR
                    if p == 0:
                        m = sv < NR
                    elif p == NPH - 1:
                        m = sv >= lo
                    else:
                        m = jnp.logical_and(sv >= lo, sv < lo + NR)
                    mi = m.astype(jnp.int32)
                    pos = plsc.cumsum(mi) - 1 + wp2[p]
                    rows = jnp.where(m, lax.shift_right_logical(pos, 7), NBW)
                    cols = jnp.where(m, jnp.bitwise_and(pos, 127), iota)
                    plsc.store_scatter(st_s.at[p], [rows, cols], sv - lo)
                    plsc.store_scatter(st_d.at[p], [rows, cols], dv)
                    n = plsc.all_reduce_population_count(m)[0]
                    new_wp.append(wp2[p] + n)
                return tuple(new_wp)

            return lax.fori_loop(0, 64, group, wp)

        z = jnp.int32(0)
        wp = lax.fori_loop(0, NB2 // 8, chunk, tuple(z for _ in range(NPH)))

        cvec = jnp.zeros((16,), jnp.int32)
        for p in range(NPH):
            nblk = lax.shift_right_logical(wp[p] + 127, 7)
            units = lax.shift_right_logical(nblk + 7, 3)

            def flush(u, cc, p=p):
                pltpu.sync_copy(st_s.at[p, pl.ds(u * 8, 8)],
                                bsrc_hbm.at[w, p, pl.ds(u * 8, 8)])
                pltpu.sync_copy(st_d.at[p, pl.ds(u * 8, 8)],
                                bdst_hbm.at[w, p, pl.ds(u * 8, 8)])
                return cc

            lax.fori_loop(0, units, flush, 0)
            cvec = cvec + jnp.where(iota == p, nblk, 0)
        cntv[0, pl.ds(0, 16)] = cvec
        pltpu.sync_copy(cntv, cnt_hbm.at[pl.ds(8 * w, 8)])

    return pl.kernel(
        body,
        out_type=(
            jax.ShapeDtypeStruct((NW, NPH, NBW, K), jnp.int32),
            jax.ShapeDtypeStruct((NW, NPH, NBW, K), jnp.int32),
            jax.ShapeDtypeStruct((NW * 8, 128), jnp.int32),
        ),
        mesh=_mesh(),
        compiler_params=pltpu.CompilerParams(needs_layout_passes=False),
        scratch_types=[
            pltpu.VMEM((8, K), jnp.int32),
            pltpu.VMEM((8, K), jnp.int32),
            pltpu.VMEM((NPH, NBW + 1, K), jnp.int32),
            pltpu.VMEM((NPH, NBW + 1, K), jnp.int32),
            pltpu.VMEM((8, 128), jnp.int32),
        ],
    )(src2, dst2, izeros, itrash)


def _zero_acc(z_hbm, acc, s):
    @pl.when(s < NS - 1)
    def _():
        pltpu.sync_copy(z_hbm, acc.at[pl.ds(s * ZMAIN, ZMAIN)])

    @pl.when(s == NS - 1)
    def _():
        pltpu.sync_copy(z_hbm.at[pl.ds(0, ZLAST)],
                        acc.at[pl.ds((NS - 1) * ZMAIN, ZLAST)])


def _copy_out(acc, out, s, base):
    @pl.when(s < NS - 1)
    def _():
        pltpu.sync_copy(acc.at[pl.ds(s * ZMAIN, ZMAIN)],
                        out.at[pl.ds(base + s * ZMAIN, ZMAIN)])

    @pl.when(s == NS - 1)
    def _():
        pltpu.sync_copy(acc.at[pl.ds((NS - 1) * ZMAIN, ZLAST)],
                        out.at[pl.ds(base + (NS - 1) * ZMAIN, ZLAST)])


def _deg_kernel(dst_idx, ones_rows, zrows):
    def body(dst_hbm, ones_hbm, z_hbm, out, idx_d, ones_v, acc):
        c = lax.axis_index("c")
        s = lax.axis_index("s")

        @pl.when(c == 0)
        def _():
            pltpu.sync_copy(ones_hbm, ones_v)
            _zero_acc(z_hbm, acc, s)
            plsc.subcore_barrier()

            def chunk(t, carry):
                pltpu.sync_copy(dst_hbm.at[s, pl.ds(t * 8, 8)], idx_d)

                def step(j, c2):
                    pltpu.sync_copy(ones_v, acc.at[idx_d.at[j]], add=True)
                    return c2

                lax.fori_loop(0, 8, step, 0)
                return carry

            lax.fori_loop(0, NB // 8, chunk, 0)
            plsc.subcore_barrier()
            _copy_out(acc, out, s, 0)

    return pl.kernel(
        body,
        out_type=jax.ShapeDtypeStruct((ACC_ROWS, 128), jnp.float32),
        mesh=_mesh(),
        scratch_types=[
            pltpu.VMEM((8, K), jnp.int32),
            pltpu.VMEM((K, 128), jnp.float32),
            pltpu.VMEM_SHARED((ACC_ROWS, 128), jnp.float32),
        ],
    )(dst_idx, ones_rows, zrows)


def _edge_apply(table, bsrc, bdst, counts, zrows, two_workers):
    def body(tab_hbm, bs_hbm, bd_hbm, cnt_hbm, z_hbm, out,
             idx_s, idx_d, buf, slab, acc, gsem, ssem):
        c = lax.axis_index("c")
        s = lax.axis_index("s")
        _zero_acc(z_hbm, acc, s)

        def read_counts(w):
            pltpu.sync_copy(cnt_hbm.at[pl.ds(8 * w, 8)], idx_s)
            cv = idx_s[0, pl.ds(0, 16)]
            return [cv[p] for p in range(NPH)]

        if two_workers:
            wqs = [2 * s, 2 * s + 1]
        else:
            wqs = [c * NS + s]
        nblks = [read_counts(w) for w in wqs]

        for p in range(NPH):
            rlo = p * NR
            tb = (c * N + rlo) if two_workers else rlo

            @pl.when(s < NS - 1)
            def _(tb=tb):
                pltpu.sync_copy(tab_hbm.at[pl.ds(tb + s * 128, 128)],
                                slab.at[pl.ds(s * 128, 128)])

            @pl.when(s == NS - 1)
            def _(tb=tb):
                pltpu.sync_copy(tab_hbm.at[pl.ds(tb + 1920, 80)],
                                slab.at[pl.ds(1920, 80)])

            plsc.subcore_barrier()

            for wloc, wq in enumerate(wqs):
                nblk = nblks[wloc][p]
                units = lax.shift_right_logical(nblk + 7, 3)

                def unit(t, cc, p=p, wq=wq):
                    pltpu.sync_copy(bs_hbm.at[wq, p, pl.ds(t * 8, 8)], idx_s)
                    pltpu.sync_copy(bd_hbm.at[wq, p, pl.ds(t * 8, 8)], idx_d)

                    def fire(j, b):
                        for q in range(8):
                            pltpu.async_copy(
                                slab.at[idx_s.at[j, pl.ds(q * 16, 16)]],
                                buf.at[b, pl.ds(q * 16, 16)], gsem.at[b, q])

                    def gwait(j, b):
                        for q in range(8):
                            pltpu.make_async_copy(
                                slab.at[idx_s.at[j, pl.ds(q * 16, 16)]],
                                buf.at[b, pl.ds(q * 16, 16)],
                                gsem.at[b, q]).wait()

                    inner = jnp.minimum(8, nblk - t * 8)
                    fire(0, 0)

                    @pl.when(inner >= 2)
                    def _():
                        fire(1, 1)

                    def step(j, c2):
                        b = lax.rem(j, 2)
                        o = 1 - b
                        gwait(j, b)
                        pltpu.async_copy(buf.at[b], acc.at[idx_d.at[j]],
                                         ssem.at[b], add=True)

                        @pl.when(jnp.logical_and(j >= 1, j < inner - 1))
                        def _():
                            pltpu.make_async_copy(buf.at[o],
                                                  acc.at[idx_d.at[j]],
                                                  ssem.at[o]).wait()
                            fire(j + 1, o)

                        return c2

                    lax.fori_loop(0, inner, step, 0)
                    pltpu.make_async_copy(buf.at[0],
                                          acc.at[idx_d.at[0]],
                                          ssem.at[lax.rem(inner - 1, 2)]
                                          ).wait()

                    @pl.when(inner >= 2)
                    def _():
                        pltpu.make_async_copy(buf.at[0],
                                              acc.at[idx_d.at[0]],
                                              ssem.at[lax.rem(inner, 2)]
                                              ).wait()
                    return cc

                lax.fori_loop(0, units, unit, 0)

            plsc.subcore_barrier()

        _copy_out(acc, out, s, c * ACC_ROWS)

    return pl.kernel(
        body,
        out_type=jax.ShapeDtypeStruct((NC * ACC_ROWS, 128), jnp.float32),
        mesh=_mesh(),
        scratch_types=[
            pltpu.VMEM((8, K), jnp.int32),
            pltpu.VMEM((8, K), jnp.int32),
            pltpu.VMEM((2, K, 128), jnp.float32),
            pltpu.VMEM_SHARED((NR, 128), jnp.float32),
            pltpu.VMEM_SHARED((ACC_ROWS, 128), jnp.float32),
            pltpu.SemaphoreType.DMA((2, 8)),
            pltpu.SemaphoreType.DMA((2,)),
        ],
    )(table, bsrc, bdst, counts, zrows)


def _tc1(x, w1, deg):
    def body(x_ref, w_ref, d_ref, o_ref):
        xw = jnp.dot(x_ref[...], w_ref[...], preferred_element_type=jnp.float32)
        d = lax.rsqrt(d_ref[:, 0:1])
        t = xw * d
        o_ref[0] = t[:, :128]
        o_ref[1] = t[:, 128:]

    return pl.pallas_call(
        body,
        grid=(G,),
        in_specs=[
            pl.BlockSpec((R, D_IN), lambda i: (i, 0)),
            pl.BlockSpec((D_IN, D_H), lambda i: (0, 0)),
            pl.BlockSpec((R, 128), lambda i: (i, 0)),
        ],
        out_specs=pl.BlockSpec((2, R, 128), lambda i: (0, i, 0)),
        out_shape=jax.ShapeDtypeStruct((2, N, 128), jnp.float32),
    )(x, w1, deg)


def _tc2(agg1, deg, b1, w2):
    def body(a_ref, d_ref, b_ref, w_ref, h_ref, t_ref):
        d = lax.rsqrt(d_ref[:, 0:1])
        h0 = jnp.maximum((a_ref[0] * d + b_ref[:, :128]) * BNC, 0.0)
        h1 = jnp.maximum((a_ref[1] * d + b_ref[:, 128:]) * BNC, 0.0)
        h_ref[0] = h0
        h_ref[1] = h1
        hcat = jnp.concatenate([h0, h1], axis=1)
        t = jnp.dot(hcat, w_ref[...], preferred_element_type=jnp.float32) * d
        t_ref[0] = t[:, :128]
        t_ref[1] = t[:, 128:]

    return pl.pallas_call(
        body,
        grid=(G,),
        in_specs=[
            pl.BlockSpec((2, R, 128), lambda i: (0, i, 0)),
            pl.BlockSpec((R, 128), lambda i: (i, 0)),
            pl.BlockSpec((1, D_H), lambda i: (0, 0)),
            pl.BlockSpec((D_H, D_H), lambda i: (0, 0)),
        ],
        out_specs=[
            pl.BlockSpec((2, R, 128), lambda i: (0, i, 0)),
            pl.BlockSpec((2, R, 128), lambda i: (0, i, 0)),
        ],
        out_shape=[
            jax.ShapeDtypeStruct((2, N, 128), jnp.float32),
            jax.ShapeDtypeStruct((2, N, 128), jnp.float32),
        ],
    )(agg1, deg, b1, w2)


def _tc3(agg2, h1, deg, b2, wmv):
    def body(a_ref, h_ref, d_ref, b_ref, w_ref, t_ref):
        d = lax.rsqrt(d_ref[:, 0:1])
        g0 = jnp.maximum((a_ref[0] * d + b_ref[:, :128]) * BNC, 0.0) + h_ref[0]
        g1 = jnp.maximum((a_ref[1] * d + b_ref[:, 128:]) * BNC, 0.0) + h_ref[1]
        h2 = jnp.concatenate([g0, g1], axis=1)
        t = jnp.dot(h2, w_ref[...], preferred_element_type=jnp.float32) * d
        t_ref[...] = t

    return pl.pallas_call(
        body,
        grid=(G,),
        in_specs=[
            pl.BlockSpec((2, R, 128), lambda i: (0, i, 0)),
            pl.BlockSpec((2, R, 128), lambda i: (0, i, 0)),
            pl.BlockSpec((R, 128), lambda i: (i, 0)),
            pl.BlockSpec((1, D_H), lambda i: (0, 0)),
            pl.BlockSpec((D_H, 128), lambda i: (0, 0)),
        ],
        out_specs=pl.BlockSpec((R, 128), lambda i: (i, 0)),
        out_shape=jax.ShapeDtypeStruct((N, 128), jnp.float32),
    )(agg2, h1, deg, b2, wmv)


def _tc4(agg3, deg, bm, bv, noise):
    def body(a_ref, d_ref, bm_ref, bv_ref, n_ref, qz_ref, qm_ref, qs_ref):
        d = lax.rsqrt(d_ref[:, 0:1])
        t = (a_ref[0] + a_ref[1]) * d
        qm = t[:, :D_L] + bm_ref[...]
        qs = t[:, D_L:] + bv_ref[...]
        std = jax.nn.softplus(qs) + 1e-6
        qz_ref[...] = qm + std * n_ref[...]
        qm_ref[...] = qm
        qs_ref[...] = qs

    return pl.pallas_call(
        body,
        grid=(G,),
        in_specs=[
            pl.BlockSpec((2, R, 128), lambda i: (0, i, 0)),
            pl.BlockSpec((R, 128), lambda i: (i, 0)),
            pl.BlockSpec((1, D_L), lambda i: (0, 0)),
            pl.BlockSpec((1, D_L), lambda i: (0, 0)),
            pl.BlockSpec((R, D_L), lambda i: (i, 0)),
        ],
        out_specs=[
            pl.BlockSpec((R, D_L), lambda i: (i, 0)),
            pl.BlockSpec((R, D_L), lambda i: (i, 0)),
            pl.BlockSpec((R, D_L), lambda i: (i, 0)),
        ],
        out_shape=[
            jax.ShapeDtypeStruct((N, D_L), jnp.float32),
            jax.ShapeDtypeStruct((N, D_L), jnp.float32),
            jax.ShapeDtypeStruct((N, D_L), jnp.float32),
        ],
    )(agg3, deg, bm, bv, noise)


def kernel(x, edge_index, W1, b1, W2, b2, Wm, bm, Wv, bv, noise):
    sl = jnp.arange(N, dtype=edge_index.dtype)
    src = jnp.concatenate([edge_index[0], sl])
    dst = jnp.concatenate([edge_index[1], sl])
    e_tot = src.shape[0]
    pad = E_PAD - e_tot
    srcp = jnp.concatenate([src, jnp.zeros((pad,), jnp.int32)])
    dstp = jnp.concatenate([dst, jnp.full((pad,), N, jnp.int32)])

    src2 = srcp.reshape(NW, NB2, K)
    dst2 = dstp.reshape(NW, NB2, K)
    dst16 = dstp.reshape(NS, NB, K)

    zrows = jnp.zeros((ZMAIN, 128), jnp.float32)
    ones_rows = jnp.ones((K, 128), jnp.float32)
    izeros = jnp.zeros((NBW, K), jnp.int32)
    itrash = jnp.full((NBW, K), N, jnp.int32)

    bsrc, bdst, cnts = _prep_kernel(src2, dst2, izeros, itrash)
    deg = _deg_kernel(dst16, ones_rows, zrows)

    t1 = _tc1(x, W1, deg)
    agg1 = _edge_apply(t1.reshape(NC * N, 128), bsrc, bdst, cnts, zrows, True)
    h1, t2 = _tc2(agg1.reshape(2, ACC_ROWS, 128), deg,
                  b1.reshape(1, D_H), W2)
    agg2 = _edge_apply(t2.reshape(NC * N, 128), bsrc, bdst, cnts, zrows, True)
    wmv = jnp.concatenate([Wm, Wv], axis=1)
    t3 = _tc3(agg2.reshape(2, ACC_ROWS, 128), h1, deg,
              b2.reshape(1, D_H), wmv)
    agg3 = _edge_apply(t3, bsrc, bdst, cnts, zrows, False)
    q_z, q_m, q_s = _tc4(agg3.reshape(2, ACC_ROWS, 128), deg,
                         bm.reshape(1, D_L), bv.reshape(1, D_L), noise)
    return (q_z, q_m, q_s)


# speedup vs baseline: 1.0348x; 1.0348x over previous
import jax
import jax.numpy as jnp
from jax import lax
from jax.experimental import pallas as pl
from jax.experimental.pallas import tpu as pltpu
from jax.experimental.pallas import tpu_sc as plsc

N = 10000
D_IN = 128
D_H = 256
D_L = 64

NC = 2
NS = 16
NW = NC * NS
K = 128
NB = 176
NB2 = NB // 2
E_PAD = NW * NB2 * K
NPH = 5
NR = 2000
NBW = 88
ACC_ROWS = 10008
ZMAIN = 632
ZLAST = 528

R = 1000
G = N // R
BNC = 0.9999950000374997


def _mesh():
    return plsc.VectorSubcoreMesh(core_axis_name="c", subcore_axis_name="s")


def _prep_kernel(src2, dst2, izeros, itrash):
    def body(src_hbm, dst_hbm, iz_hbm, it_hbm, bsrc_hbm, bdst_hbm, cnt_hbm,
             in_s, in_d, st_s, st_d, cntv):
        c = lax.axis_index("c")
        s = lax.axis_index("s")
        w = c * NS + s
        for p in range(NPH):
            pltpu.sync_copy(iz_hbm, st_s.at[p, pl.ds(0, NBW)])
            pltpu.sync_copy(it_hbm, st_d.at[p, pl.ds(0, NBW)])
        iota = lax.broadcasted_iota(jnp.int32, (16,), 0)

        def chunk(t, wp):
            pltpu.sync_copy(src_hbm.at[w, pl.ds(t * 8, 8)], in_s)
            pltpu.sync_copy(dst_hbm.at[w, pl.ds(t * 8, 8)], in_d)

            def group(g, wp2):
                r = lax.shift_right_logical(g, 3)
                col = jnp.bitwise_and(g, 7) * 16
                sv = in_s[r, pl.ds(col, 16)]
                dv = in_d[r, pl.ds(col, 16)]
                new_wp = []
                for p in range(NPH):
                    lo = p * NR
                    if p == 0:
                        m = sv < NR
                    elif p == NPH - 1:
                        m = sv >= lo
                    else:
                        m = jnp.logical_and(sv >= lo, sv < lo + NR)
                    mi = m.astype(jnp.int32)
                    pos = plsc.cumsum(mi) - 1 + wp2[p]
                    rows = jnp.where(m, lax.shift_right_logical(pos, 7), NBW)
                    cols = jnp.where(m, jnp.bitwise_and(pos, 127), iota)
                    plsc.store_scatter(st_s.at[p], [rows, cols], sv - lo)
                    plsc.store_scatter(st_d.at[p], [rows, cols], dv)
                    n = plsc.all_reduce_population_count(m)[0]
                    new_wp.append(wp2[p] + n)
                return tuple(new_wp)

            return lax.fori_loop(0, 64, group, wp)

        z = jnp.int32(0)
        wp = lax.fori_loop(0, NB2 // 8, chunk, tuple(z for _ in range(NPH)))

        cvec = jnp.zeros((16,), jnp.int32)
        for p in range(NPH):
            nblk = lax.shift_right_logical(wp[p] + 127, 7)
            units = lax.shift_right_logical(nblk + 7, 3)

            def flush(u, cc, p=p):
                pltpu.sync_copy(st_s.at[p, pl.ds(u * 8, 8)],
                                bsrc_hbm.at[w, p, pl.ds(u * 8, 8)])
                pltpu.sync_copy(st_d.at[p, pl.ds(u * 8, 8)],
                                bdst_hbm.at[w, p, pl.ds(u * 8, 8)])
                return cc

            lax.fori_loop(0, units, flush, 0)
            cvec = cvec + jnp.where(iota == p, nblk, 0)
        cntv[0, pl.ds(0, 16)] = cvec
        pltpu.sync_copy(cntv, cnt_hbm.at[pl.ds(8 * w, 8)])

    return pl.kernel(
        body,
        out_type=(
            jax.ShapeDtypeStruct((NW, NPH, NBW, K), jnp.int32),
            jax.ShapeDtypeStruct((NW, NPH, NBW, K), jnp.int32),
            jax.ShapeDtypeStruct((NW * 8, 128), jnp.int32),
        ),
        mesh=_mesh(),
        compiler_params=pltpu.CompilerParams(needs_layout_passes=False),
        scratch_types=[
            pltpu.VMEM((8, K), jnp.int32),
            pltpu.VMEM((8, K), jnp.int32),
            pltpu.VMEM((NPH, NBW + 1, K), jnp.int32),
            pltpu.VMEM((NPH, NBW + 1, K), jnp.int32),
            pltpu.VMEM((8, 128), jnp.int32),
        ],
    )(src2, dst2, izeros, itrash)


def _zero_acc(z_hbm, acc, s):
    @pl.when(s < NS - 1)
    def _():
        pltpu.sync_copy(z_hbm, acc.at[pl.ds(s * ZMAIN, ZMAIN)])

    @pl.when(s == NS - 1)
    def _():
        pltpu.sync_copy(z_hbm.at[pl.ds(0, ZLAST)],
                        acc.at[pl.ds((NS - 1) * ZMAIN, ZLAST)])


def _copy_out(acc, out, s, base):
    @pl.when(s < NS - 1)
    def _():
        pltpu.sync_copy(acc.at[pl.ds(s * ZMAIN, ZMAIN)],
                        out.at[pl.ds(base + s * ZMAIN, ZMAIN)])

    @pl.when(s == NS - 1)
    def _():
        pltpu.sync_copy(acc.at[pl.ds((NS - 1) * ZMAIN, ZLAST)],
                        out.at[pl.ds(base + (NS - 1) * ZMAIN, ZLAST)])


def _deg_kernel(dst_idx, ones_rows, zrows):
    def body(dst_hbm, ones_hbm, z_hbm, out, idx_d, ones_v, acc):
        c = lax.axis_index("c")
        s = lax.axis_index("s")
        w = c * NS + s
        pltpu.sync_copy(ones_hbm, ones_v)
        _zero_acc(z_hbm, acc, s)
        plsc.subcore_barrier()

        def chunk(t, carry):
            pltpu.sync_copy(dst_hbm.at[w, pl.ds(t * 8, 8)], idx_d)

            def step(j, c2):
                pltpu.sync_copy(ones_v, acc.at[idx_d.at[j]], add=True)
                return c2

            lax.fori_loop(0, 8, step, 0)
            return carry

        lax.fori_loop(0, NB2 // 8, chunk, 0)
        plsc.subcore_barrier()
        _copy_out(acc, out, s, c * ACC_ROWS)

    return pl.kernel(
        body,
        out_type=jax.ShapeDtypeStruct((NC * ACC_ROWS, 128), jnp.float32),
        mesh=_mesh(),
        scratch_types=[
            pltpu.VMEM((8, K), jnp.int32),
            pltpu.VMEM((K, 128), jnp.float32),
            pltpu.VMEM_SHARED((ACC_ROWS, 128), jnp.float32),
        ],
    )(dst_idx, ones_rows, zrows)


def _edge_apply(table, bsrc, bdst, counts, zrows, two_workers):
    def body(tab_hbm, bs_hbm, bd_hbm, cnt_hbm, z_hbm, out,
             idx_s, idx_d, buf, slab, acc, gsem, ssem):
        c = lax.axis_index("c")
        s = lax.axis_index("s")
        _zero_acc(z_hbm, acc, s)

        def read_counts(w):
            pltpu.sync_copy(cnt_hbm.at[pl.ds(8 * w, 8)], idx_s)
            cv = idx_s[0, pl.ds(0, 16)]
            return [cv[p] for p in range(NPH)]

        if two_workers:
            wqs = [2 * s, 2 * s + 1]
        else:
            wqs = [c * NS + s]
        nblks = [read_counts(w) for w in wqs]

        for p in range(NPH):
            rlo = p * NR
            tb = (c * N + rlo) if two_workers else rlo

            @pl.when(s < NS - 1)
            def _(tb=tb):
                pltpu.sync_copy(tab_hbm.at[pl.ds(tb + s * 128, 128)],
                                slab.at[pl.ds(s * 128, 128)])

            @pl.when(s == NS - 1)
            def _(tb=tb):
                pltpu.sync_copy(tab_hbm.at[pl.ds(tb + 1920, 80)],
                                slab.at[pl.ds(1920, 80)])

            plsc.subcore_barrier()

            for wloc, wq in enumerate(wqs):
                nblk = nblks[wloc][p]
                units = lax.shift_right_logical(nblk + 7, 3)

                def unit(t, cc, p=p, wq=wq):
                    pltpu.sync_copy(bs_hbm.at[wq, p, pl.ds(t * 8, 8)], idx_s)
                    pltpu.sync_copy(bd_hbm.at[wq, p, pl.ds(t * 8, 8)], idx_d)

                    def fire(j, b):
                        for q in range(8):
                            pltpu.async_copy(
                                slab.at[idx_s.at[j, pl.ds(q * 16, 16)]],
                                buf.at[b, pl.ds(q * 16, 16)], gsem.at[b, q])

                    def gwait(j, b):
                        for q in range(8):
                            pltpu.make_async_copy(
                                slab.at[idx_s.at[j, pl.ds(q * 16, 16)]],
                                buf.at[b, pl.ds(q * 16, 16)],
                                gsem.at[b, q]).wait()

                    inner = jnp.minimum(8, nblk - t * 8)
                    fire(0, 0)

                    @pl.when(inner >= 2)
                    def _():
                        fire(1, 1)

                    def step(j, c2):
                        b = lax.rem(j, 2)
                        o = 1 - b
                        gwait(j, b)
                        pltpu.async_copy(buf.at[b], acc.at[idx_d.at[j]],
                                         ssem.at[b], add=True)

                        @pl.when(jnp.logical_and(j >= 1, j < inner - 1))
                        def _():
                            pltpu.make_async_copy(buf.at[o],
                                                  acc.at[idx_d.at[j]],
                                                  ssem.at[o]).wait()
                            fire(j + 1, o)

                        return c2

                    lax.fori_loop(0, inner, step, 0)
                    pltpu.make_async_copy(buf.at[0],
                                          acc.at[idx_d.at[0]],
                                          ssem.at[lax.rem(inner - 1, 2)]
                                          ).wait()

                    @pl.when(inner >= 2)
                    def _():
                        pltpu.make_async_copy(buf.at[0],
                                              acc.at[idx_d.at[0]],
                                              ssem.at[lax.rem(inner, 2)]
                                              ).wait()
                    return cc

                lax.fori_loop(0, units, unit, 0)

            plsc.subcore_barrier()

        _copy_out(acc, out, s, c * ACC_ROWS)

    return pl.kernel(
        body,
        out_type=jax.ShapeDtypeStruct((NC * ACC_ROWS, 128), jnp.float32),
        mesh=_mesh(),
        scratch_types=[
            pltpu.VMEM((8, K), jnp.int32),
            pltpu.VMEM((8, K), jnp.int32),
            pltpu.VMEM((2, K, 128), jnp.float32),
            pltpu.VMEM_SHARED((NR, 128), jnp.float32),
            pltpu.VMEM_SHARED((ACC_ROWS, 128), jnp.float32),
            pltpu.SemaphoreType.DMA((2, 8)),
            pltpu.SemaphoreType.DMA((2,)),
        ],
    )(table, bsrc, bdst, counts, zrows)


def _tc1(x, w1, deg):
    def body(x_ref, w_ref, d_ref, o_ref):
        xw = jnp.dot(x_ref[...], w_ref[...], preferred_element_type=jnp.float32)
        d = lax.rsqrt(d_ref[0][:, 0:1] + d_ref[1][:, 0:1])
        t = xw * d
        o_ref[0] = t[:, :128]
        o_ref[1] = t[:, 128:]

    return pl.pallas_call(
        body,
        grid=(G,),
        in_specs=[
            pl.BlockSpec((R, D_IN), lambda i: (i, 0)),
            pl.BlockSpec((D_IN, D_H), lambda i: (0, 0)),
            pl.BlockSpec((2, R, 128), lambda i: (0, i, 0)),
        ],
        out_specs=pl.BlockSpec((2, R, 128), lambda i: (0, i, 0)),
        out_shape=jax.ShapeDtypeStruct((2, N, 128), jnp.float32),
    )(x, w1, deg)


def _tc2(agg1, deg, b1, w2):
    def body(a_ref, d_ref, b_ref, w_ref, h_ref, t_ref):
        d = lax.rsqrt(d_ref[0][:, 0:1] + d_ref[1][:, 0:1])
        h0 = jnp.maximum((a_ref[0] * d + b_ref[:, :128]) * BNC, 0.0)
        h1 = jnp.maximum((a_ref[1] * d + b_ref[:, 128:]) * BNC, 0.0)
        h_ref[0] = h0
        h_ref[1] = h1
        hcat = jnp.concatenate([h0, h1], axis=1)
        t = jnp.dot(hcat, w_ref[...], preferred_element_type=jnp.float32) * d
        t_ref[0] = t[:, :128]
        t_ref[1] = t[:, 128:]

    return pl.pallas_call(
        body,
        grid=(G,),
        in_specs=[
            pl.BlockSpec((2, R, 128), lambda i: (0, i, 0)),
            pl.BlockSpec((2, R, 128), lambda i: (0, i, 0)),
            pl.BlockSpec((1, D_H), lambda i: (0, 0)),
            pl.BlockSpec((D_H, D_H), lambda i: (0, 0)),
        ],
        out_specs=[
            pl.BlockSpec((2, R, 128), lambda i: (0, i, 0)),
            pl.BlockSpec((2, R, 128), lambda i: (0, i, 0)),
        ],
        out_shape=[
            jax.ShapeDtypeStruct((2, N, 128), jnp.float32),
            jax.ShapeDtypeStruct((2, N, 128), jnp.float32),
        ],
    )(agg1, deg, b1, w2)


def _tc3(agg2, h1, deg, b2, wmv):
    def body(a_ref, h_ref, d_ref, b_ref, w_ref, t_ref):
        d = lax.rsqrt(d_ref[0][:, 0:1] + d_ref[1][:, 0:1])
        g0 = jnp.maximum((a_ref[0] * d + b_ref[:, :128]) * BNC, 0.0) + h_ref[0]
        g1 = jnp.maximum((a_ref[1] * d + b_ref[:, 128:]) * BNC, 0.0) + h_ref[1]
        h2 = jnp.concatenate([g0, g1], axis=1)
        t = jnp.dot(h2, w_ref[...], preferred_element_type=jnp.float32) * d
        t_ref[...] = t

    return pl.pallas_call(
        body,
        grid=(G,),
        in_specs=[
            pl.BlockSpec((2, R, 128), lambda i: (0, i, 0)),
            pl.BlockSpec((2, R, 128), lambda i: (0, i, 0)),
            pl.BlockSpec((2, R, 128), lambda i: (0, i, 0)),
            pl.BlockSpec((1, D_H), lambda i: (0, 0)),
            pl.BlockSpec((D_H, 128), lambda i: (0, 0)),
        ],
        out_specs=pl.BlockSpec((R, 128), lambda i: (i, 0)),
        out_shape=jax.ShapeDtypeStruct((N, 128), jnp.float32),
    )(agg2, h1, deg, b2, wmv)


def _tc4(agg3, deg, bm, bv, noise):
    def body(a_ref, d_ref, bm_ref, bv_ref, n_ref, qz_ref, qm_ref, qs_ref):
        d = lax.rsqrt(d_ref[0][:, 0:1] + d_ref[1][:, 0:1])
        t = (a_ref[0] + a_ref[1]) * d
        qm = t[:, :D_L] + bm_ref[...]
        qs = t[:, D_L:] + bv_ref[...]
        std = jax.nn.softplus(qs) + 1e-6
        qz_ref[...] = qm + std * n_ref[...]
        qm_ref[...] = qm
        qs_ref[...] = qs

    return pl.pallas_call(
        body,
        grid=(G,),
        in_specs=[
            pl.BlockSpec((2, R, 128), lambda i: (0, i, 0)),
            pl.BlockSpec((2, R, 128), lambda i: (0, i, 0)),
            pl.BlockSpec((1, D_L), lambda i: (0, 0)),
            pl.BlockSpec((1, D_L), lambda i: (0, 0)),
            pl.BlockSpec((R, D_L), lambda i: (i, 0)),
        ],
        out_specs=[
            pl.BlockSpec((R, D_L), lambda i: (i, 0)),
            pl.BlockSpec((R, D_L), lambda i: (i, 0)),
            pl.BlockSpec((R, D_L), lambda i: (i, 0)),
        ],
        out_shape=[
            jax.ShapeDtypeStruct((N, D_L), jnp.float32),
            jax.ShapeDtypeStruct((N, D_L), jnp.float32),
            jax.ShapeDtypeStruct((N, D_L), jnp.float32),
        ],
    )(agg3, deg, bm, bv, noise)


def kernel(x, edge_index, W1, b1, W2, b2, Wm, bm, Wv, bv, noise):
    sl = jnp.arange(N, dtype=edge_index.dtype)
    src = jnp.concatenate([edge_index[0], sl])
    dst = jnp.concatenate([edge_index[1], sl])
    e_tot = src.shape[0]
    pad = E_PAD - e_tot
    srcp = jnp.concatenate([src, jnp.zeros((pad,), jnp.int32)])
    dstp = jnp.concatenate([dst, jnp.full((pad,), N, jnp.int32)])

    src2 = srcp.reshape(NW, NB2, K)
    dst2 = dstp.reshape(NW, NB2, K)

    zrows = jnp.zeros((ZMAIN, 128), jnp.float32)
    ones_rows = jnp.ones((K, 128), jnp.float32)
    izeros = jnp.zeros((NBW, K), jnp.int32)
    itrash = jnp.full((NBW, K), N, jnp.int32)

    bsrc, bdst, cnts = _prep_kernel(src2, dst2, izeros, itrash)
    deg = _deg_kernel(dst2, ones_rows, zrows).reshape(2, ACC_ROWS, 128)

    t1 = _tc1(x, W1, deg)
    agg1 = _edge_apply(t1.reshape(NC * N, 128), bsrc, bdst, cnts, zrows, True)
    h1, t2 = _tc2(agg1.reshape(2, ACC_ROWS, 128), deg,
                  b1.reshape(1, D_H), W2)
    agg2 = _edge_apply(t2.reshape(NC * N, 128), bsrc, bdst, cnts, zrows, True)
    wmv = jnp.concatenate([Wm, Wv], axis=1)
    t3 = _tc3(agg2.reshape(2, ACC_ROWS, 128), h1, deg,
              b2.reshape(1, D_H), wmv)
    agg3 = _edge_apply(t3, bsrc, bdst, cnts, zrows, False)
    q_z, q_m, q_s = _tc4(agg3.reshape(2, ACC_ROWS, 128), deg,
                         bm.reshape(1, D_L), bv.reshape(1, D_L), noise)
    return (q_z, q_m, q_s)
